# baseline (speedup 1.0000x reference)
import functools

import jax
import jax.numpy as jnp
from jax import lax
from jax.experimental import pallas as pl
from jax.experimental.pallas import tpu as pltpu
from jax.experimental.pallas import tpu_sc as plsc

_VOCAB = 1000000
_DEMO_VOCAB = 1000
_H = 64
_B, _L = 4096, 200
_N = _B * _L
_NW = 32
_PER_W = _N // _NW
_C = 128
_NCHUNK = _PER_W // _C
_NLANES = 16
_UNROLL = 8


def _sc_body(wid_hbm, age_hbm, bmi_hbm, cyc_hbm, wt_hbm, demo_hbm, gb_hbm,
             out_hbm, idxw, idxa, idxb, idxc, rows, demo, gb_v, sem):
    worker = lax.axis_index("s") * 2 + lax.axis_index("c")
    pltpu.sync_copy(demo_hbm, demo)
    pltpu.sync_copy(gb_hbm, gb_v)
    lanes = lax.iota(jnp.int32, _NLANES)
    woff = worker * _PER_W

    def chunk(i, carry):
        base = woff + i * _C
        pltpu.sync_copy(wid_hbm.at[pl.ds(base, _C)], idxw)
        pltpu.sync_copy(age_hbm.at[pl.ds(base, _C)], idxa)
        pltpu.sync_copy(bmi_hbm.at[pl.ds(base, _C)], idxb)
        pltpu.sync_copy(cyc_hbm.at[pl.ds(base, _C)], idxc)
        pltpu.async_copy(wt_hbm.at[idxw], rows, sem).wait()

        for g in range(_C // _NLANES):
            tok = lanes + (g * _NLANES)
            a0 = idxa[pl.ds(g * _NLANES, _NLANES)] * _H
            b0 = idxb[pl.ds(g * _NLANES, _NLANES)] * _H
            c0 = idxc[pl.ds(g * _NLANES, _NLANES)] * _H

            def pass_a(j, sc):
                s, s2 = sc
                h0 = j * _UNROLL
                for k in range(_UNROLL):
                    h = h0 + k
                    col = jnp.full((_NLANES,), h, jnp.int32)
                    x = (plsc.load_gather(rows, [tok, col])
                         + plsc.load_gather(demo, [a0 + h])
                         + plsc.load_gather(demo, [b0 + h])
                         + plsc.load_gather(demo, [c0 + h]))
                    plsc.store_scatter(rows, [tok, col], x)
                    s = s + x
                    s2 = s2 + x * x
                return (s, s2)

            zero = jnp.zeros((_NLANES,), jnp.float32)
            s, s2 = lax.fori_loop(0, _H // _UNROLL, pass_a, (zero, zero))
            mean = s * (1.0 / _H)
            var = s2 * (1.0 / _H) - mean * mean
            v = var + 1e-12
            y = plsc.bitcast(
                jnp.int32(0x5F3759DF) - (plsc.bitcast(v, jnp.int32) >> 1),
                jnp.float32)
            for _ in range(3):
                y = y * (1.5 - 0.5 * v * y * y)
            rstd = y

            def pass_b(j, carry):
                h0 = j * _UNROLL
                for k in range(_UNROLL):
                    h = h0 + k
                    col = jnp.full((_NLANES,), h, jnp.int32)
                    x = plsc.load_gather(rows, [tok, col])
                    out = (x - mean) * rstd * gb_v[h] + gb_v[h + _H]
                    plsc.store_scatter(rows, [tok, col], out)
                return carry

            lax.fori_loop(0, _H // _UNROLL, pass_b, 0)

        pltpu.sync_copy(rows, out_hbm.at[pl.ds(base, _C)])
        return carry

    lax.fori_loop(0, _NCHUNK, chunk, 0)


@jax.jit
def kernel(word_ids, age_ids, bmi_ids, cycle_len_ids, word_table, demo_table,
           gamma, beta):
    wid = word_ids.reshape(_N).astype(jnp.int32)
    age = age_ids.reshape(_N).astype(jnp.int32)
    bmi = bmi_ids.reshape(_N).astype(jnp.int32)
    cyc = cycle_len_ids.reshape(_N).astype(jnp.int32)
    demo_flat = demo_table.reshape(_DEMO_VOCAB * _H)
    gb = jnp.repeat(
        jnp.concatenate([gamma, beta]).astype(jnp.float32)[:, None],
        _NLANES, axis=1)

    mesh = plsc.VectorSubcoreMesh(core_axis_name="c", subcore_axis_name="s")
    run = pl.kernel(
        _sc_body,
        out_type=jax.ShapeDtypeStruct((_N, _H), jnp.float32),
        mesh=mesh,
        scratch_types=[
            pltpu.VMEM((_C,), jnp.int32),
            pltpu.VMEM((_C,), jnp.int32),
            pltpu.VMEM((_C,), jnp.int32),
            pltpu.VMEM((_C,), jnp.int32),
            pltpu.VMEM((_C, _H), jnp.float32),
            pltpu.VMEM((_DEMO_VOCAB * _H,), jnp.float32),
            pltpu.VMEM((2 * _H, _NLANES), jnp.float32),
            pltpu.SemaphoreType.DMA,
        ],
        compiler_params=pltpu.CompilerParams(
            needs_layout_passes=False, use_tc_tiling_on_sc=False),
    )
    out = run(wid, age, bmi, cyc, word_table, demo_flat, gb)
    return out.reshape(_B, _L, _H)


# speedup vs baseline: 1.4592x; 1.4592x over previous
import functools

import jax
import jax.numpy as jnp
from jax import lax
from jax.experimental import pallas as pl
from jax.experimental.pallas import tpu as pltpu
from jax.experimental.pallas import tpu_sc as plsc

_VOCAB = 1000000
_DEMO_VOCAB = 1000
_H = 64
_B, _L = 4096, 200
_N = _B * _L
_NW = 32
_PER_W = _N // _NW
_C = 128
_NCHUNK = _PER_W // _C
_NLANES = 16
_UNROLL = 8


def _sc_body(wid_hbm, age_hbm, bmi_hbm, cyc_hbm, wt_hbm, demo_hbm, gb_hbm,
             out_hbm, idxw, idxa, idxb, idxc, rows, xbuf, obuf, demo, gb_v,
             sem):
    worker = lax.axis_index("s") * 2 + lax.axis_index("c")
    pltpu.sync_copy(demo_hbm, demo)
    pltpu.sync_copy(gb_hbm, gb_v)
    lanes = lax.iota(jnp.int32, _NLANES)
    woff = worker * _PER_W

    def chunk(i, carry):
        base = woff + i * _C
        pltpu.sync_copy(wid_hbm.at[pl.ds(base, _C)], idxw)
        pltpu.sync_copy(age_hbm.at[pl.ds(base, _C)], idxa)
        pltpu.sync_copy(bmi_hbm.at[pl.ds(base, _C)], idxb)
        pltpu.sync_copy(cyc_hbm.at[pl.ds(base, _C)], idxc)
        pltpu.async_copy(wt_hbm.at[idxw], rows, sem).wait()

        for g in range(_C // _NLANES):
            tok = lanes + (g * _NLANES)
            a0 = idxa[pl.ds(g * _NLANES, _NLANES)] * _H
            b0 = idxb[pl.ds(g * _NLANES, _NLANES)] * _H
            c0 = idxc[pl.ds(g * _NLANES, _NLANES)] * _H

            zero = jnp.zeros((_NLANES,), jnp.float32)

            @plsc.parallel_loop(0, _H, step=1, unroll=_UNROLL,
                                carry=(zero, zero))
            def pass_a(h, sc):
                s, s2 = sc
                col = jnp.full((_NLANES,), h, jnp.int32)
                x = (plsc.load_gather(rows, [tok, col])
                     + plsc.load_gather(demo, [a0 + h])
                     + plsc.load_gather(demo, [b0 + h])
                     + plsc.load_gather(demo, [c0 + h]))
                plsc.store_scatter(xbuf, [tok, col], x)
                return (s + x, s2 + x * x)

            s, s2 = pass_a
            mean = s * (1.0 / _H)
            var = s2 * (1.0 / _H) - mean * mean
            v = var + 1e-12
            y = plsc.bitcast(
                jnp.int32(0x5F3759DF) - (plsc.bitcast(v, jnp.int32) >> 1),
                jnp.float32)
            for _ in range(3):
                y = y * (1.5 - 0.5 * v * y * y)
            rstd = y

            @plsc.parallel_loop(0, _H, step=1, unroll=_UNROLL)
            def pass_b(h):
                col = jnp.full((_NLANES,), h, jnp.int32)
                x = plsc.load_gather(xbuf, [tok, col])
                out = (x - mean) * rstd * gb_v[h] + gb_v[h + _H]
                plsc.store_scatter(obuf, [tok, col], out)

            del pass_b

        pltpu.sync_copy(obuf, out_hbm.at[pl.ds(base, _C)])
        return carry

    lax.fori_loop(0, _NCHUNK, chunk, 0)


@jax.jit
def kernel(word_ids, age_ids, bmi_ids, cycle_len_ids, word_table, demo_table,
           gamma, beta):
    wid = word_ids.reshape(_N).astype(jnp.int32)
    age = age_ids.reshape(_N).astype(jnp.int32)
    bmi = bmi_ids.reshape(_N).astype(jnp.int32)
    cyc = cycle_len_ids.reshape(_N).astype(jnp.int32)
    demo_flat = demo_table.reshape(_DEMO_VOCAB * _H)
    gb = jnp.repeat(
        jnp.concatenate([gamma, beta]).astype(jnp.float32)[:, None],
        _NLANES, axis=1)

    mesh = plsc.VectorSubcoreMesh(core_axis_name="c", subcore_axis_name="s")
    run = pl.kernel(
        _sc_body,
        out_type=jax.ShapeDtypeStruct((_N, _H), jnp.float32),
        mesh=mesh,
        scratch_types=[
            pltpu.VMEM((_C,), jnp.int32),
            pltpu.VMEM((_C,), jnp.int32),
            pltpu.VMEM((_C,), jnp.int32),
            pltpu.VMEM((_C,), jnp.int32),
            pltpu.VMEM((_C, _H), jnp.float32),
            pltpu.VMEM((_C, _H), jnp.float32),
            pltpu.VMEM((_C, _H), jnp.float32),
            pltpu.VMEM((_DEMO_VOCAB * _H,), jnp.float32),
            pltpu.VMEM((2 * _H, _NLANES), jnp.float32),
            pltpu.SemaphoreType.DMA,
        ],
        compiler_params=pltpu.CompilerParams(
            needs_layout_passes=False, use_tc_tiling_on_sc=False),
    )
    out = run(wid, age, bmi, cyc, word_table, demo_flat, gb)
    return out.reshape(_B, _L, _H)


# speedup vs baseline: 1.5188x; 1.0409x over previous
import functools

import jax
import jax.numpy as jnp
from jax import lax
from jax.experimental import pallas as pl
from jax.experimental.pallas import tpu as pltpu
from jax.experimental.pallas import tpu_sc as plsc

_VOCAB = 1000000
_DEMO_VOCAB = 1000
_H = 64
_B, _L = 4096, 200
_N = _B * _L
_NW = 32
_PER_W = _N // _NW
_C = 128
_NCHUNK = _PER_W // _C
_NLANES = 16
_UNROLL = 8


def _sc_body(wid_hbm, age_hbm, bmi_hbm, cyc_hbm, wt_hbm, demo_hbm, gb_hbm,
             out_hbm, idxw, idxa, idxb, idxc, rows, xbuf, obuf, demo, gb_v,
             sem):
    worker = lax.axis_index("s") * 2 + lax.axis_index("c")
    pltpu.sync_copy(demo_hbm, demo)
    pltpu.sync_copy(gb_hbm, gb_v)
    lanes = lax.iota(jnp.int32, _NLANES)
    woff = worker * _PER_W

    def chunk(i, carry):
        base = woff + i * _C
        pltpu.sync_copy(wid_hbm.at[pl.ds(base, _C)], idxw)
        pltpu.sync_copy(age_hbm.at[pl.ds(base, _C)], idxa)
        pltpu.sync_copy(bmi_hbm.at[pl.ds(base, _C)], idxb)
        pltpu.sync_copy(cyc_hbm.at[pl.ds(base, _C)], idxc)
        pltpu.async_copy(wt_hbm.at[idxw], rows, sem).wait()

        zrow = jnp.zeros((_NLANES,), jnp.int32)
        for g in range(_C // _NLANES):
            t0 = (lanes + (g * _NLANES)) * _H
            a0 = idxa[pl.ds(g * _NLANES, _NLANES)] * _H
            b0 = idxb[pl.ds(g * _NLANES, _NLANES)] * _H
            c0 = idxc[pl.ds(g * _NLANES, _NLANES)] * _H

            zero = jnp.zeros((_NLANES,), jnp.float32)

            @plsc.parallel_loop(0, _H, step=1, unroll=_UNROLL,
                                carry=(zero, zero))
            def pass_a(h, sc):
                s, s2 = sc
                flat = t0 + h
                x = (plsc.load_gather(rows, [zrow, flat])
                     + plsc.load_gather(demo, [a0 + h])
                     + plsc.load_gather(demo, [b0 + h])
                     + plsc.load_gather(demo, [c0 + h]))
                plsc.store_scatter(xbuf, [zrow, flat], x)
                return (s + x, s2 + x * x)

            s, s2 = pass_a
            mean = s * (1.0 / _H)
            var = s2 * (1.0 / _H) - mean * mean
            v = var + 1e-12
            y = plsc.bitcast(
                jnp.int32(0x5F3759DF) - (plsc.bitcast(v, jnp.int32) >> 1),
                jnp.float32)
            for _ in range(3):
                y = y * (1.5 - 0.5 * v * y * y)
            rstd = y

            @plsc.parallel_loop(0, _H, step=1, unroll=_UNROLL)
            def pass_b(h):
                flat = t0 + h
                x = plsc.load_gather(xbuf, [zrow, flat])
                out = (x - mean) * rstd * gb_v[h] + gb_v[h + _H]
                plsc.store_scatter(obuf, [zrow, flat], out)

            del pass_b

        pltpu.sync_copy(obuf, out_hbm.at[pl.ds(base, _C)])
        return carry

    lax.fori_loop(0, _NCHUNK, chunk, 0)


@jax.jit
def kernel(word_ids, age_ids, bmi_ids, cycle_len_ids, word_table, demo_table,
           gamma, beta):
    wid = word_ids.reshape(_N).astype(jnp.int32)
    age = age_ids.reshape(_N).astype(jnp.int32)
    bmi = bmi_ids.reshape(_N).astype(jnp.int32)
    cyc = cycle_len_ids.reshape(_N).astype(jnp.int32)
    demo_flat = demo_table.reshape(_DEMO_VOCAB * _H)
    gb = jnp.repeat(
        jnp.concatenate([gamma, beta]).astype(jnp.float32)[:, None],
        _NLANES, axis=1)

    mesh = plsc.VectorSubcoreMesh(core_axis_name="c", subcore_axis_name="s")
    run = pl.kernel(
        _sc_body,
        out_type=jax.ShapeDtypeStruct((_N, _H), jnp.float32),
        mesh=mesh,
        scratch_types=[
            pltpu.VMEM((_C,), jnp.int32),
            pltpu.VMEM((_C,), jnp.int32),
            pltpu.VMEM((_C,), jnp.int32),
            pltpu.VMEM((_C,), jnp.int32),
            pltpu.VMEM((_C, _H), jnp.float32),
            pltpu.VMEM((_C, _H), jnp.float32),
            pltpu.VMEM((_C, _H), jnp.float32),
            pltpu.VMEM((_DEMO_VOCAB * _H,), jnp.float32),
            pltpu.VMEM((2 * _H, _NLANES), jnp.float32),
            pltpu.SemaphoreType.DMA,
        ],
        compiler_params=pltpu.CompilerParams(
            needs_layout_passes=False, use_tc_tiling_on_sc=False),
    )
    out = run(wid, age, bmi, cyc, word_table, demo_flat, gb)
    return out.reshape(_B, _L, _H)


# speedup vs baseline: 3.8706x; 2.5485x over previous
import functools

import jax
import jax.numpy as jnp
from jax import lax
from jax.experimental import pallas as pl
from jax.experimental.pallas import tpu as pltpu
from jax.experimental.pallas import tpu_sc as plsc

_VOCAB = 1000000
_DEMO_VOCAB = 1000
_H = 64
_B, _L = 4096, 200
_N = _B * _L
_NW = 32
_PER_W = _N // _NW
_C = 128
_NCHUNK = _PER_W // _C
_NLANES = 16
_UNROLL = 8


def _sc_body(wid_hbm, age_hbm, bmi_hbm, cyc_hbm, wt_hbm, demo_hbm, gb_hbm,
             out_hbm, idxw, idxa, idxb, idxc, rows, xbuf, obuf, demo, gb_v,
             sem):
    worker = lax.axis_index("s") * 2 + lax.axis_index("c")
    pltpu.sync_copy(demo_hbm, demo)
    pltpu.sync_copy(gb_hbm, gb_v)
    lanes = lax.iota(jnp.int32, _NLANES)
    woff = worker * _PER_W

    def chunk(i, carry):
        base = woff + i * _C
        pltpu.sync_copy(wid_hbm.at[pl.ds(base, _C)], idxw)
        pltpu.sync_copy(age_hbm.at[pl.ds(base, _C)], idxa)
        pltpu.sync_copy(bmi_hbm.at[pl.ds(base, _C)], idxb)
        pltpu.sync_copy(cyc_hbm.at[pl.ds(base, _C)], idxc)
        pltpu.async_copy(wt_hbm.at[idxw], rows, sem).wait()

        zrow = jnp.zeros((_NLANES,), jnp.int32)
        for g in range(_C // _NLANES):
            t0 = (lanes + (g * _NLANES)) * _H
            a0 = idxa[pl.ds(g * _NLANES, _NLANES)] * _H
            b0 = idxb[pl.ds(g * _NLANES, _NLANES)] * _H
            c0 = idxc[pl.ds(g * _NLANES, _NLANES)] * _H

            zero = jnp.zeros((_NLANES,), jnp.float32)

            @plsc.parallel_loop(0, _H, step=1, unroll=_UNROLL,
                                carry=(zero, zero))
            def pass_a(h, sc):
                s, s2 = sc
                gcol = (h + lanes) & (_H - 1)
                flat = t0 + gcol
                x = (plsc.load_gather(rows, [zrow, flat])
                     + plsc.load_gather(demo, [a0 + gcol])
                     + plsc.load_gather(demo, [b0 + gcol])
                     + plsc.load_gather(demo, [c0 + gcol]))
                plsc.store_scatter(xbuf, [zrow, flat], x)
                return (s + x, s2 + x * x)

            s, s2 = pass_a
            mean = s * (1.0 / _H)
            var = s2 * (1.0 / _H) - mean * mean
            v = var + 1e-12
            y = plsc.bitcast(
                jnp.int32(0x5F3759DF) - (plsc.bitcast(v, jnp.int32) >> 1),
                jnp.float32)
            for _ in range(3):
                y = y * (1.5 - 0.5 * v * y * y)
            rstd = y

            @plsc.parallel_loop(0, _H, step=1, unroll=_UNROLL)
            def pass_b(h):
                gcol = (h + lanes) & (_H - 1)
                flat = t0 + gcol
                x = plsc.load_gather(xbuf, [zrow, flat])
                gv = plsc.load_gather(gb_v, [gcol])
                bv = plsc.load_gather(gb_v, [gcol + _H])
                out = (x - mean) * rstd * gv + bv
                plsc.store_scatter(obuf, [zrow, flat], out)

            del pass_b

        pltpu.sync_copy(obuf, out_hbm.at[pl.ds(base, _C)])
        return carry

    lax.fori_loop(0, _NCHUNK, chunk, 0)


@jax.jit
def kernel(word_ids, age_ids, bmi_ids, cycle_len_ids, word_table, demo_table,
           gamma, beta):
    wid = word_ids.reshape(_N).astype(jnp.int32)
    age = age_ids.reshape(_N).astype(jnp.int32)
    bmi = bmi_ids.reshape(_N).astype(jnp.int32)
    cyc = cycle_len_ids.reshape(_N).astype(jnp.int32)
    demo_flat = demo_table.reshape(_DEMO_VOCAB * _H)
    gb = jnp.concatenate([gamma, beta]).astype(jnp.float32)

    mesh = plsc.VectorSubcoreMesh(core_axis_name="c", subcore_axis_name="s")
    run = pl.kernel(
        _sc_body,
        out_type=jax.ShapeDtypeStruct((_N, _H), jnp.float32),
        mesh=mesh,
        scratch_types=[
            pltpu.VMEM((_C,), jnp.int32),
            pltpu.VMEM((_C,), jnp.int32),
            pltpu.VMEM((_C,), jnp.int32),
            pltpu.VMEM((_C,), jnp.int32),
            pltpu.VMEM((_C, _H), jnp.float32),
            pltpu.VMEM((_C, _H), jnp.float32),
            pltpu.VMEM((_C, _H), jnp.float32),
            pltpu.VMEM((_DEMO_VOCAB * _H,), jnp.float32),
            pltpu.VMEM((2 * _H,), jnp.float32),
            pltpu.SemaphoreType.DMA,
        ],
        compiler_params=pltpu.CompilerParams(
            needs_layout_passes=False, use_tc_tiling_on_sc=False),
    )
    out = run(wid, age, bmi, cyc, word_table, demo_flat, gb)
    return out.reshape(_B, _L, _H)


# speedup vs baseline: 4.9971x; 1.2910x over previous
import jax
import jax.numpy as jnp
from jax import lax
from jax.experimental import pallas as pl
from jax.experimental.pallas import tpu as pltpu
from jax.experimental.pallas import tpu_sc as plsc

_VOCAB = 1000000
_DEMO_VOCAB = 1000
_H = 64
_B, _L = 4096, 200
_N = _B * _L
_NW = 32
_C = 128
_NCHUNK = _N // (_NW * _C)
_NCHT = _N // _C
_NLANES = 16
_UNROLL = 8


def _compute_chunk(didxb, rowsb, xbuf, obufb, demo, gb_v, lanes, zrow):
    for g in range(_C // _NLANES):
        t0 = (lanes + (g * _NLANES)) * _H
        a0 = didxb[0, pl.ds(g * _NLANES, _NLANES)] * _H
        b0 = didxb[1, pl.ds(g * _NLANES, _NLANES)] * _H
        c0 = didxb[2, pl.ds(g * _NLANES, _NLANES)] * _H

        zero = jnp.zeros((_NLANES,), jnp.float32)

        @plsc.parallel_loop(0, _H, step=1, unroll=_UNROLL,
                            carry=(zero, zero))
        def pass_a(h, sc):
            s, s2 = sc
            gcol = (h + lanes) & (_H - 1)
            flat = t0 + gcol
            x = (plsc.load_gather(rowsb, [zrow, flat])
                 + plsc.load_gather(demo, [a0 + gcol])
                 + plsc.load_gather(demo, [b0 + gcol])
                 + plsc.load_gather(demo, [c0 + gcol]))
            plsc.store_scatter(xbuf, [zrow, flat], x)
            return (s + x, s2 + x * x)

        s, s2 = pass_a
        mean = s * (1.0 / _H)
        var = s2 * (1.0 / _H) - mean * mean
        v = var + 1e-12
        y = plsc.bitcast(
            jnp.int32(0x5F3759DF) - (plsc.bitcast(v, jnp.int32) >> 1),
            jnp.float32)
        for _ in range(3):
            y = y * (1.5 - 0.5 * v * y * y)
        rstd = y

        @plsc.parallel_loop(0, _H, step=1, unroll=_UNROLL)
        def pass_b(h):
            gcol = (h + lanes) & (_H - 1)
            flat = t0 + gcol
            x = plsc.load_gather(xbuf, [zrow, flat])
            gv = plsc.load_gather(gb_v, [gcol])
            bv = plsc.load_gather(gb_v, [gcol + _H])
            out = (x - mean) * rstd * gv + bv
            plsc.store_scatter(obufb, [zrow, flat], out)

        del pass_b


def _sc_body(widx_hbm, didx_hbm, wt_hbm, demo_hbm, gb_hbm, out_hbm,
             widx0, widx1, didx0, didx1, rows0, rows1, xbuf, obuf0, obuf1,
             demo, gb_v,
             iwsem0, iwsem1, idsem0, idsem1, gsem0, gsem1, osem0, osem1):
    worker = lax.axis_index("s") * 2 + lax.axis_index("c")
    pltpu.sync_copy(demo_hbm, demo)
    pltpu.sync_copy(gb_hbm, gb_v)
    lanes = lax.iota(jnp.int32, _NLANES)
    zrow = jnp.zeros((_NLANES,), jnp.int32)
    c0 = worker * _NCHUNK

    widx = (widx0, widx1)
    didx = (didx0, didx1)
    rows = (rows0, rows1)
    obuf = (obuf0, obuf1)
    iwsem = (iwsem0, iwsem1)
    idsem = (idsem0, idsem1)
    gsem = (gsem0, gsem1)
    osem = (osem0, osem1)

    pltpu.sync_copy(widx_hbm.at[c0], widx0)
    pltpu.async_copy(wt_hbm.at[widx0], rows0, gsem0)
    pltpu.async_copy(widx_hbm.at[c0 + 1], widx1, iwsem1)
    pltpu.async_copy(didx_hbm.at[c0], didx0, idsem0)
    pltpu.async_copy(didx_hbm.at[c0 + 1], didx1, idsem1)

    def step(j, carry):
        for b in range(2):
            nb = 1 - b
            i = j * 2 + b

            @pl.when(i < _NCHUNK - 1)
            def _():
                pltpu.make_async_copy(
                    widx_hbm.at[c0], widx[nb], iwsem[nb]).wait()
                pltpu.async_copy(wt_hbm.at[widx[nb]], rows[nb], gsem[nb])

            pltpu.make_async_copy(
                wt_hbm.at[pl.ds(0, _C)], rows[b], gsem[b]).wait()

            @pl.when(i < _NCHUNK - 2)
            def _():
                pltpu.async_copy(
                    widx_hbm.at[c0 + i + 2], widx[b], iwsem[b])

            @pl.when(i >= 2)
            def _():
                pltpu.make_async_copy(
                    obuf[b], out_hbm.at[pl.ds(0, _C)], osem[b]).wait()

            pltpu.make_async_copy(
                didx_hbm.at[c0], didx[b], idsem[b]).wait()

            _compute_chunk(didx[b], rows[b], xbuf, obuf[b], demo, gb_v,
                           lanes, zrow)

            @pl.when(i < _NCHUNK - 2)
            def _():
                pltpu.async_copy(
                    didx_hbm.at[c0 + i + 2], didx[b], idsem[b])

            pltpu.async_copy(
                obuf[b], out_hbm.at[pl.ds((c0 + i) * _C, _C)], osem[b])
        return carry

    lax.fori_loop(0, _NCHUNK // 2, step, 0)
    pltpu.make_async_copy(obuf0, out_hbm.at[pl.ds(0, _C)], osem0).wait()
    pltpu.make_async_copy(obuf1, out_hbm.at[pl.ds(0, _C)], osem1).wait()


@jax.jit
def kernel(word_ids, age_ids, bmi_ids, cycle_len_ids, word_table, demo_table,
           gamma, beta):
    widx = word_ids.reshape(_NCHT, _C).astype(jnp.int32)
    didx = (jnp.stack([age_ids.reshape(_N), bmi_ids.reshape(_N),
                       cycle_len_ids.reshape(_N)])
            .astype(jnp.int32).reshape(3, _NCHT, _C).transpose(1, 0, 2))
    demo_flat = demo_table.reshape(_DEMO_VOCAB * _H)
    gb = jnp.concatenate([gamma, beta]).astype(jnp.float32)

    mesh = plsc.VectorSubcoreMesh(core_axis_name="c", subcore_axis_name="s")
    run = pl.kernel(
        _sc_body,
        out_type=jax.ShapeDtypeStruct((_N, _H), jnp.float32),
        mesh=mesh,
        scratch_types=[
            pltpu.VMEM((_C,), jnp.int32),
            pltpu.VMEM((_C,), jnp.int32),
            pltpu.VMEM((3, _C), jnp.int32),
            pltpu.VMEM((3, _C), jnp.int32),
            pltpu.VMEM((_C, _H), jnp.float32),
            pltpu.VMEM((_C, _H), jnp.float32),
            pltpu.VMEM((_C, _H), jnp.float32),
            pltpu.VMEM((_C, _H), jnp.float32),
            pltpu.VMEM((_C, _H), jnp.float32),
            pltpu.VMEM((_DEMO_VOCAB * _H,), jnp.float32),
            pltpu.VMEM((2 * _H,), jnp.float32),
            pltpu.SemaphoreType.DMA,
            pltpu.SemaphoreType.DMA,
            pltpu.SemaphoreType.DMA,
            pltpu.SemaphoreType.DMA,
            pltpu.SemaphoreType.DMA,
            pltpu.SemaphoreType.DMA,
            pltpu.SemaphoreType.DMA,
            pltpu.SemaphoreType.DMA,
        ],
        compiler_params=pltpu.CompilerParams(
            needs_layout_passes=False, use_tc_tiling_on_sc=False),
    )
    out = run(widx, didx, word_table, demo_flat, gb)
    return out.reshape(_B, _L, _H)
